# baseline (speedup 1.0000x reference)
import functools

import jax
import jax.numpy as jnp
from jax import lax
from jax.experimental import pallas as pl
from jax.experimental.pallas import tpu as pltpu
from jax.experimental.pallas import tpu_sc as plsc

VOCAB_TILE = 2048


def _gather_sc(table2, idx_hi):
    batch = idx_hi.shape[0]
    _, width = table2.shape
    info = plsc.get_sparse_core_info()
    num_workers = info.num_cores * info.num_subcores
    b_per_w = batch // num_workers

    mesh = plsc.VectorSubcoreMesh(core_axis_name="c", subcore_axis_name="s")

    @functools.partial(
        pl.kernel,
        mesh=mesh,
        out_type=jax.ShapeDtypeStruct((batch, width), jnp.float32),
        scratch_types=[
            pltpu.VMEM((b_per_w,), jnp.int32),
            pltpu.VMEM((b_per_w, width), jnp.float32),
            pltpu.SemaphoreType.DMA,
        ],
    )
    def gather_kernel(table_hbm, idx_hbm, out_hbm, idx_v, rows_v, sem):
        wid = lax.axis_index("s") * info.num_cores + lax.axis_index("c")
        base = wid * b_per_w
        pltpu.sync_copy(idx_hbm.at[pl.ds(base, b_per_w)], idx_v)
        pltpu.async_copy(table_hbm.at[idx_v], rows_v, sem).wait()
        pltpu.sync_copy(rows_v, out_hbm.at[pl.ds(base, b_per_w)])

    return gather_kernel(table2, idx_hi)


def _proj_kernel(e2_ref, p_ref, w_ref, b_ref, out_ref):
    dim = w_ref.shape[1]
    p = p_ref[...]
    e = e2_ref[:, :dim] * (1.0 - p) + e2_ref[:, dim:] * p
    out_ref[...] = (
        lax.dot_general(
            e,
            w_ref[...],
            dimension_numbers=(((1,), (1,)), ((), ())),
            preferred_element_type=jnp.float32,
        )
        + b_ref[...]
    )


def kernel(x, emb_table, W, b):
    batch = x.shape[0]
    vocab, dim = W.shape
    xi = x.astype(jnp.int32)
    table2 = emb_table.reshape(emb_table.shape[0] // 2, 2 * dim)
    e2 = _gather_sc(table2, xi >> 1)
    parity = (xi & 1).astype(jnp.float32).reshape(batch, 1)
    b2 = b.reshape(1, vocab)
    grid = pl.cdiv(vocab, VOCAB_TILE)
    logits = pl.pallas_call(
        _proj_kernel,
        grid=(grid,),
        in_specs=[
            pl.BlockSpec((batch, 2 * dim), lambda i: (0, 0)),
            pl.BlockSpec((batch, 1), lambda i: (0, 0)),
            pl.BlockSpec((VOCAB_TILE, dim), lambda i: (i, 0)),
            pl.BlockSpec((1, VOCAB_TILE), lambda i: (0, i)),
        ],
        out_specs=pl.BlockSpec((batch, VOCAB_TILE), lambda i: (0, i)),
        out_shape=jax.ShapeDtypeStruct((batch, vocab), jnp.float32),
    )(e2, parity, W, b2)
    return logits
